# baseline (speedup 1.0000x reference)
import functools

import jax
import jax.numpy as jnp
from jax import lax
from jax.experimental import pallas as pl
from jax.experimental.pallas import tpu as pltpu
from jax.experimental.pallas import tpu_sc as plsc

_N = 32768
_K = 2048
_ROWS = 128
_NW = 32
_RPW = _ROWS // _NW
_SL = _N // 16

_INT_MIN = -2147483648
_MANT = 0x7FFFFFFF
_INF_BITS = 0x7F800000
_Y_LO = 0x3FB33333
_PIVOT = 1.4
_UNROLL = 8
_FUSE_UNROLL = 4
_CW = 8192
_W2 = 2048
_B_SW = 16


def _decode(y_vec):
    bits = jnp.where(y_vec >= 0, y_vec, y_vec ^ _MANT)
    return lax.bitcast_convert_type(bits, jnp.float32)


def _sc_body(x_hbm, o_hbm, x_a, x_b, cand_a, cand_b, cand2, sem_a, sem_b):
    wid = lax.axis_index("c") * 16 + lax.axis_index("s")
    lane = lax.iota(jnp.int32, 16)
    zero_f = jnp.zeros((16,), jnp.float32)
    inf_f = jnp.full((16,), jnp.inf, jnp.float32)
    row0 = wid * _RPW

    def search_threshold(x_v, cand, m_cnt):
        fb = (m_cnt < _K + 1) | (m_cnt > _CW - 16 * _UNROLL)
        m_eff = jnp.where(fb, jnp.int32(_N), m_cnt)
        y_floor = jnp.where(fb, jnp.int32(_INT_MIN), jnp.int32(_Y_LO))

        @pl.when(jnp.logical_not(fb))
        def _():
            for u in range(_UNROLL):
                pad_idx = m_cnt + (16 * u) + lane
                plsc.store_scatter(cand, [pad_idx], inf_f,
                                   mask=pad_idx < _CW)

        r_a = m_eff - (_K + 1)
        n_bl = (m_eff + 16 * _UNROLL - 1) // (16 * _UNROLL)

        def count_below(ref, piv, n_blocks):
            def cnt_body(jj, accs):
                a0, a1 = accs
                for u in range(_UNROLL):
                    xv = ref[pl.ds((jj * _UNROLL + u) * 16, 16)]
                    hit = (xv < piv).astype(jnp.int32)
                    if u % 2 == 0:
                        a0 = a0 + hit
                    else:
                        a1 = a1 + hit
                return a0, a1

            z16 = jnp.zeros((16,), jnp.int32)
            a0, a1 = lax.fori_loop(0, n_blocks, cnt_body, (z16, z16))
            return jnp.sum(a0 + a1)

        def count1(piv):
            return lax.cond(
                fb,
                lambda: count_below(x_v, piv, _SL // _UNROLL),
                lambda: count_below(cand, piv, n_bl))

        def bit_body1(i, prefix):
            b = 31 - i
            c = prefix | lax.shift_left(jnp.int32(1), b)
            y_c = c ^ jnp.int32(_INT_MIN)
            cnt = lax.cond(
                y_c <= y_floor, lambda: jnp.int32(0),
                lambda: count1(_decode(jnp.full((16,), y_c))))
            return jnp.where(cnt <= r_a, c, prefix)

        prefix = lax.fori_loop(0, 32 - _B_SW, bit_body1, jnp.int32(0))

        lo_y = jnp.full((16,), prefix ^ jnp.int32(_INT_MIN))
        hi_y = jnp.minimum(lo_y + (1 << _B_SW), _INF_BITS)
        lo_f = _decode(lo_y)
        hi_f = _decode(hi_y)

        def win_body(jj, carry):
            off, c_lo = carry
            for u in range(_UNROLL):
                x = cand[pl.ds((jj * _UNROLL + u) * 16, 16)]
                m_lo = x < lo_f
                m_in = (x >= lo_f) & (x < hi_f)
                c_lo = c_lo + plsc.all_reduce_population_count(m_lo)[0]
                plsc.store_compressed(
                    cand2.at[pl.ds(jnp.minimum(off, _W2 - 16), 16)],
                    x, mask=m_in)
                off = off + plsc.all_reduce_population_count(m_in)[0]
            return off, c_lo

        use2 = jnp.logical_not(fb)
        w_cnt, c_lo = lax.cond(
            use2,
            lambda: lax.fori_loop(0, n_bl, win_body,
                                  (jnp.int32(0), jnp.int32(0))),
            lambda: (jnp.int32(0), jnp.int32(0)))
        use2 = use2 & (w_cnt <= _W2 - 32)

        @pl.when(use2)
        def _():
            for u in range(2):
                pad_idx = w_cnt + (16 * u) + lane
                plsc.store_scatter(cand2, [pad_idx], inf_f,
                                   mask=pad_idx < _W2)

        n_bl2 = (w_cnt + 15) // 16

        def count2_below(piv):
            def cnt_body(jj, acc):
                xv = cand2[pl.ds(jj * 16, 16)]
                return acc + (xv < piv).astype(jnp.int32)
            acc = lax.fori_loop(0, n_bl2, cnt_body,
                                jnp.zeros((16,), jnp.int32))
            return jnp.sum(acc)

        def bit_body2(i, prefix):
            b = _B_SW - 1 - i
            c = prefix | lax.shift_left(jnp.int32(1), b)
            y_c = c ^ jnp.int32(_INT_MIN)
            piv = _decode(jnp.full((16,), y_c))
            cnt = lax.cond(
                use2,
                lambda: c_lo + count2_below(piv),
                lambda: lax.cond(
                    y_c <= y_floor, lambda: jnp.int32(0),
                    lambda: count1(piv)))
            return jnp.where(cnt <= r_a, c, prefix)

        prefix = lax.fori_loop(0, _B_SW, bit_body2, prefix)
        return _decode(jnp.full((16,), prefix ^ jnp.int32(_INT_MIN)))

    for pair in range(_RPW // 2):
        row_a = row0 + 2 * pair
        row_b = row_a + 1
        if pair > 0:
            pltpu.make_async_copy(x_a, o_hbm.at[row_a - 2], sem_a).wait()
            pltpu.make_async_copy(x_b, o_hbm.at[row_b - 2], sem_b).wait()
        pltpu.async_copy(x_hbm.at[row_a], x_a, sem_a)
        pltpu.async_copy(x_hbm.at[row_b], x_b, sem_b)
        pltpu.make_async_copy(x_hbm.at[row_a], x_a, sem_a).wait()
        pltpu.make_async_copy(x_hbm.at[row_b], x_b, sem_b).wait()

        def comp_body(jj, offs):
            off_a, off_b = offs
            work = []
            for u in range(_FUSE_UNROLL):
                j = (jj * _FUSE_UNROLL + u) * 16
                xa = x_a[pl.ds(j, 16)]
                xb = x_b[pl.ds(j, 16)]
                ma = xa >= _PIVOT
                mb = xb >= _PIVOT
                pa = plsc.all_reduce_population_count(ma)[0]
                pb = plsc.all_reduce_population_count(mb)[0]
                work.append((xa, ma, pa, xb, mb, pb))
            for xa, ma, pa, xb, mb, pb in work:
                plsc.store_compressed(
                    cand_a.at[pl.ds(jnp.minimum(off_a, _CW - 16), 16)],
                    xa, mask=ma)
                plsc.store_compressed(
                    cand_b.at[pl.ds(jnp.minimum(off_b, _CW - 16), 16)],
                    xb, mask=mb)
                off_a = off_a + pa
                off_b = off_b + pb
            return off_a, off_b

        m_cnt_a, m_cnt_b = lax.fori_loop(
            0, _SL // _FUSE_UNROLL, comp_body, (jnp.int32(0), jnp.int32(0)))

        thr_a = search_threshold(x_a, cand_a, m_cnt_a)
        thr_b = search_threshold(x_b, cand_b, m_cnt_b)

        def mask_body(jj, carry):
            for u in range(_FUSE_UNROLL):
                j = (jj * _FUSE_UNROLL + u) * 16
                xa = x_a[pl.ds(j, 16)]
                xb = x_b[pl.ds(j, 16)]
                x_a[pl.ds(j, 16)] = jnp.where(xa > thr_a, xa, zero_f)
                x_b[pl.ds(j, 16)] = jnp.where(xb > thr_b, xb, zero_f)
            return carry

        lax.fori_loop(0, _SL // _FUSE_UNROLL, mask_body, 0)
        pltpu.async_copy(x_a, o_hbm.at[row_a], sem_a)
        pltpu.async_copy(x_b, o_hbm.at[row_b], sem_b)

    pltpu.make_async_copy(x_a, o_hbm.at[row0 + _RPW - 2], sem_a).wait()
    pltpu.make_async_copy(x_b, o_hbm.at[row0 + _RPW - 1], sem_b).wait()


@jax.jit
def kernel(inputs):
    f = functools.partial(
        pl.kernel,
        out_type=jax.ShapeDtypeStruct((_ROWS, _N), jnp.float32),
        mesh=plsc.VectorSubcoreMesh(core_axis_name="c", subcore_axis_name="s"),
        scratch_types=[
            pltpu.VMEM((_N,), jnp.float32),
            pltpu.VMEM((_N,), jnp.float32),
            pltpu.VMEM((_CW,), jnp.float32),
            pltpu.VMEM((_CW,), jnp.float32),
            pltpu.VMEM((_W2,), jnp.float32),
            pltpu.SemaphoreType.DMA,
            pltpu.SemaphoreType.DMA,
        ],
        compiler_params=pltpu.CompilerParams(needs_layout_passes=False),
    )(_sc_body)
    return f(inputs)


# speedup vs baseline: 1.1009x; 1.1009x over previous
import functools

import jax
import jax.numpy as jnp
from jax import lax
from jax.experimental import pallas as pl
from jax.experimental.pallas import tpu as pltpu
from jax.experimental.pallas import tpu_sc as plsc

_N = 32768
_K = 2048
_ROWS = 128
_NW = 32
_RPW = _ROWS // _NW
_SL = _N // 16

_INT_MIN = -2147483648
_MANT = 0x7FFFFFFF
_INF_BITS = 0x7F800000
_Y_LO = 0x3FB33333
_PIVOT = 1.4
_UNROLL = 8
_FUSE_UNROLL = 8
_CW = 8192
_W2 = 2048
_B_SW = 16


def _decode(y_vec):
    bits = jnp.where(y_vec >= 0, y_vec, y_vec ^ _MANT)
    return lax.bitcast_convert_type(bits, jnp.float32)


def _sc_body(x_hbm, o_hbm, x_a, x_b, cand_a, cand_b, cand2, sem_a, sem_b):
    wid = lax.axis_index("c") * 16 + lax.axis_index("s")
    lane = lax.iota(jnp.int32, 16)
    zero_f = jnp.zeros((16,), jnp.float32)
    inf_f = jnp.full((16,), jnp.inf, jnp.float32)
    row0 = wid * _RPW

    def search_threshold(x_v, cand, m_cnt):
        fb = (m_cnt < _K + 1) | (m_cnt > _CW - 16 * _UNROLL)
        m_eff = jnp.where(fb, jnp.int32(_N), m_cnt)
        y_floor = jnp.where(fb, jnp.int32(_INT_MIN), jnp.int32(_Y_LO))

        @pl.when(jnp.logical_not(fb))
        def _():
            for u in range(_UNROLL):
                pad_idx = m_cnt + (16 * u) + lane
                plsc.store_scatter(cand, [pad_idx], inf_f,
                                   mask=pad_idx < _CW)

        r_a = m_eff - (_K + 1)
        n_bl = (m_eff + 16 * _UNROLL - 1) // (16 * _UNROLL)

        def count_below(ref, piv, n_blocks):
            def cnt_body(jj, accs):
                a0, a1 = accs
                for u in range(_UNROLL):
                    xv = ref[pl.ds((jj * _UNROLL + u) * 16, 16)]
                    hit = (xv < piv).astype(jnp.int32)
                    if u % 2 == 0:
                        a0 = a0 + hit
                    else:
                        a1 = a1 + hit
                return a0, a1

            z16 = jnp.zeros((16,), jnp.int32)
            a0, a1 = lax.fori_loop(0, n_blocks, cnt_body, (z16, z16))
            return jnp.sum(a0 + a1)

        def count1(piv):
            return lax.cond(
                fb,
                lambda: count_below(x_v, piv, _SL // _UNROLL),
                lambda: count_below(cand, piv, n_bl))

        def bit_body1(i, prefix):
            b = 31 - i
            c = prefix | lax.shift_left(jnp.int32(1), b)
            y_c = c ^ jnp.int32(_INT_MIN)
            cnt = lax.cond(
                y_c <= y_floor, lambda: jnp.int32(0),
                lambda: count1(_decode(jnp.full((16,), y_c))))
            return jnp.where(cnt <= r_a, c, prefix)

        prefix = lax.fori_loop(0, 32 - _B_SW, bit_body1, jnp.int32(0))

        lo_y = jnp.full((16,), prefix ^ jnp.int32(_INT_MIN))
        hi_y = jnp.minimum(lo_y + (1 << _B_SW), _INF_BITS)
        lo_f = _decode(lo_y)
        hi_f = _decode(hi_y)

        def win_body(jj, carry):
            off, c_lo = carry
            for u in range(_UNROLL):
                x = cand[pl.ds((jj * _UNROLL + u) * 16, 16)]
                m_lo = x < lo_f
                m_in = (x >= lo_f) & (x < hi_f)
                c_lo = c_lo + plsc.all_reduce_population_count(m_lo)[0]
                plsc.store_compressed(
                    cand2.at[pl.ds(jnp.minimum(off, _W2 - 16), 16)],
                    x, mask=m_in)
                off = off + plsc.all_reduce_population_count(m_in)[0]
            return off, c_lo

        use2 = jnp.logical_not(fb)
        w_cnt, c_lo = lax.cond(
            use2,
            lambda: lax.fori_loop(0, n_bl, win_body,
                                  (jnp.int32(0), jnp.int32(0))),
            lambda: (jnp.int32(0), jnp.int32(0)))
        use2 = use2 & (w_cnt <= _W2 - 32)

        @pl.when(use2)
        def _():
            for u in range(2):
                pad_idx = w_cnt + (16 * u) + lane
                plsc.store_scatter(cand2, [pad_idx], inf_f,
                                   mask=pad_idx < _W2)

        n_bl2 = (w_cnt + 15) // 16

        def count2_below(piv):
            def cnt_body(jj, acc):
                xv = cand2[pl.ds(jj * 16, 16)]
                return acc + (xv < piv).astype(jnp.int32)
            acc = lax.fori_loop(0, n_bl2, cnt_body,
                                jnp.zeros((16,), jnp.int32))
            return jnp.sum(acc)

        def bit_body2(i, prefix):
            b = _B_SW - 1 - i
            c = prefix | lax.shift_left(jnp.int32(1), b)
            y_c = c ^ jnp.int32(_INT_MIN)
            piv = _decode(jnp.full((16,), y_c))
            cnt = lax.cond(
                use2,
                lambda: c_lo + count2_below(piv),
                lambda: lax.cond(
                    y_c <= y_floor, lambda: jnp.int32(0),
                    lambda: count1(piv)))
            return jnp.where(cnt <= r_a, c, prefix)

        prefix = lax.fori_loop(0, _B_SW, bit_body2, prefix)
        return _decode(jnp.full((16,), prefix ^ jnp.int32(_INT_MIN)))

    for pair in range(_RPW // 2):
        row_a = row0 + 2 * pair
        row_b = row_a + 1
        if pair > 0:
            pltpu.make_async_copy(x_a, o_hbm.at[row_a - 2], sem_a).wait()
            pltpu.make_async_copy(x_b, o_hbm.at[row_b - 2], sem_b).wait()
        pltpu.async_copy(x_hbm.at[row_a], x_a, sem_a)
        pltpu.async_copy(x_hbm.at[row_b], x_b, sem_b)
        pltpu.make_async_copy(x_hbm.at[row_a], x_a, sem_a).wait()
        pltpu.make_async_copy(x_hbm.at[row_b], x_b, sem_b).wait()

        def comp_body(jj, offs):
            off_a, off_b = offs
            work = []
            for u in range(_FUSE_UNROLL):
                j = (jj * _FUSE_UNROLL + u) * 16
                xa = x_a[pl.ds(j, 16)]
                xb = x_b[pl.ds(j, 16)]
                ma = xa >= _PIVOT
                mb = xb >= _PIVOT
                pa = plsc.all_reduce_population_count(ma)[0]
                pb = plsc.all_reduce_population_count(mb)[0]
                work.append((xa, ma, pa, xb, mb, pb))
            for xa, ma, pa, xb, mb, pb in work:
                plsc.store_compressed(
                    cand_a.at[pl.ds(jnp.minimum(off_a, _CW - 16), 16)],
                    xa, mask=ma)
                plsc.store_compressed(
                    cand_b.at[pl.ds(jnp.minimum(off_b, _CW - 16), 16)],
                    xb, mask=mb)
                off_a = off_a + pa
                off_b = off_b + pb
            return off_a, off_b

        m_cnt_a, m_cnt_b = lax.fori_loop(
            0, _SL // _FUSE_UNROLL, comp_body, (jnp.int32(0), jnp.int32(0)))

        thr_a = search_threshold(x_a, cand_a, m_cnt_a)
        thr_b = search_threshold(x_b, cand_b, m_cnt_b)

        def mask_body(jj, carry):
            for u in range(_FUSE_UNROLL):
                j = (jj * _FUSE_UNROLL + u) * 16
                xa = x_a[pl.ds(j, 16)]
                xb = x_b[pl.ds(j, 16)]
                x_a[pl.ds(j, 16)] = jnp.where(xa > thr_a, xa, zero_f)
                x_b[pl.ds(j, 16)] = jnp.where(xb > thr_b, xb, zero_f)
            return carry

        lax.fori_loop(0, _SL // _FUSE_UNROLL, mask_body, 0)
        pltpu.async_copy(x_a, o_hbm.at[row_a], sem_a)
        pltpu.async_copy(x_b, o_hbm.at[row_b], sem_b)

    pltpu.make_async_copy(x_a, o_hbm.at[row0 + _RPW - 2], sem_a).wait()
    pltpu.make_async_copy(x_b, o_hbm.at[row0 + _RPW - 1], sem_b).wait()


@jax.jit
def kernel(inputs):
    f = functools.partial(
        pl.kernel,
        out_type=jax.ShapeDtypeStruct((_ROWS, _N), jnp.float32),
        mesh=plsc.VectorSubcoreMesh(core_axis_name="c", subcore_axis_name="s"),
        scratch_types=[
            pltpu.VMEM((_N,), jnp.float32),
            pltpu.VMEM((_N,), jnp.float32),
            pltpu.VMEM((_CW,), jnp.float32),
            pltpu.VMEM((_CW,), jnp.float32),
            pltpu.VMEM((_W2,), jnp.float32),
            pltpu.SemaphoreType.DMA,
            pltpu.SemaphoreType.DMA,
        ],
        compiler_params=pltpu.CompilerParams(needs_layout_passes=False),
    )(_sc_body)
    return f(inputs)
